# baseline (speedup 1.0000x reference)
import jax
import jax.numpy as jnp
from jax import lax
from jax.experimental import pallas as pl
from jax.experimental.pallas import tpu as pltpu
from jax.experimental.pallas import tpu_sc as plsc

_N = 100000
_LL = 10
_ED = 16
_OUT_D = _ED + 2 + _LL

_NW = 32
_SPAN = 3136
_CNT = 1568
_VI = _CNT // 16


def _body(an_hbm, nu_hbm, l_hbm, ll_hbm, tab_hbm, out_hbm,
          idx_v, l_v, ll_v, tab_v, out_v, sem):
    wid = lax.axis_index("s") * 2 + lax.axis_index("c")
    t = wid * (_N // _NW)
    wbase = pl.multiple_of(jnp.minimum(t - lax.rem(t, 8), _N - _SPAN), 8)

    pltpu.sync_copy(tab_hbm, tab_v)
    for k in range(_SPAN // _CNT):
      base = pl.multiple_of(wbase + k * _CNT, 8)
      pltpu.sync_copy(an_hbm.at[pl.ds(base, _CNT)], idx_v)
      pltpu.sync_copy(l_hbm.at[pl.ds(base, _CNT)], l_v)
      pltpu.sync_copy(ll_hbm.at[pl.ds(base, _CNT)], ll_v)

      d_nu = pltpu.async_copy(
          nu_hbm.at[pl.ds(base, _CNT)], out_v.at[:, pl.ds(_ED, 1)], sem
      )

      iota = lax.iota(jnp.int32, 16)
      cols = [jnp.full((16,), j, jnp.int32) for j in range(_OUT_D)]
      llc = [jnp.full((16,), j, jnp.int32) for j in range(_LL)]

      def vec_body(i, _):
          rows = iota + i * 16
          an = plsc.load_gather(idx_v, [rows])
          for j in range(_ED):
              v = plsc.load_gather(tab_v, [an, cols[j]])
              plsc.store_scatter(out_v, [rows, cols[j]], v)
          lv = plsc.load_gather(l_v, [rows])
          plsc.store_scatter(out_v, [rows, cols[_ED + 1]], lv)
          for j in range(_LL):
              v = plsc.load_gather(ll_v, [rows, llc[j]])
              plsc.store_scatter(out_v, [rows, cols[_ED + 2 + j]], v)
          return _

      lax.fori_loop(0, _VI, vec_body, None)

      d_nu.wait()

      pltpu.sync_copy(out_v, out_hbm.at[pl.ds(base, _CNT)])


@jax.jit
def _node_feat(an, nu2, l2, ll, tab):
    mesh = plsc.VectorSubcoreMesh(core_axis_name="c", subcore_axis_name="s")
    run = pl.kernel(
        _body,
        out_type=jax.ShapeDtypeStruct((_N, _OUT_D), jnp.float32),
        mesh=mesh,
        scratch_types=[
            pltpu.VMEM((_CNT,), jnp.int32),
            pltpu.VMEM((_CNT,), jnp.float32),
            pltpu.VMEM((_CNT, _LL), jnp.float32),
            pltpu.VMEM((10, _ED), jnp.float32),
            pltpu.VMEM((_CNT, _OUT_D), jnp.float32),
            pltpu.SemaphoreType.DMA,
        ],
        compiler_params=pltpu.CompilerParams(
            use_tc_tiling_on_sc=False, needs_layout_passes=False
        ),
    )
    return run(an, nu2, l2, ll, tab)


def kernel(atomic_numbers, max_nu, max_l, l_list, emb_table):
    an = atomic_numbers.astype(jnp.int32)
    nu2 = max_nu.reshape(_N, 1)
    return _node_feat(an, nu2, max_l, l_list, emb_table)


# speedup vs baseline: 1.7483x; 1.7483x over previous
import jax
import jax.numpy as jnp
from jax import lax
from jax.experimental import pallas as pl
from jax.experimental.pallas import tpu as pltpu
from jax.experimental.pallas import tpu_sc as plsc

_N = 100000
_LL = 10
_ED = 16
_OUT_D = _ED + 2 + _LL

_NW = 32
_NC = 2
_CNT = 1568
_SPAN = _NC * _CNT


def _body(an_hbm, nu_hbm, l_hbm, ll_hbm, tab_hbm, out_hbm,
          idx_v, nu_v, l_v, ll_v, tab_v, out_vs, sem, sem_o):
    wid = lax.axis_index("s") * 2 + lax.axis_index("c")
    t = wid * (_N // _NW)
    wbase = pl.multiple_of(jnp.minimum(t - lax.rem(t, 8), _N - _SPAN), 8)

    pltpu.sync_copy(tab_hbm, tab_v)

    iota = lax.iota(jnp.int32, 16)
    cols = [jnp.full((16,), j, jnp.int32) for j in range(_OUT_D)]
    llc = [jnp.full((16,), j, jnp.int32) for j in range(_LL)]

    out_descs = [None] * _NC
    for k in range(_NC):
        base = pl.multiple_of(wbase + k * _CNT, 8)
        out_v = out_vs[k]

        ds_in = [
            pltpu.async_copy(an_hbm.at[pl.ds(base, _CNT)], idx_v, sem),
            pltpu.async_copy(nu_hbm.at[pl.ds(base, _CNT)], nu_v, sem),
            pltpu.async_copy(l_hbm.at[pl.ds(base, _CNT)], l_v, sem),
            pltpu.async_copy(ll_hbm.at[pl.ds(base, _CNT)], ll_v, sem),
        ]
        for d in ds_in:
            d.wait()

        @plsc.parallel_loop(0, _CNT, step=16, unroll=2)
        def vec_body(i):
            rows = iota + i
            an = plsc.load_gather(idx_v, [rows])
            for j in range(_ED):
                v = plsc.load_gather(tab_v, [an, cols[j]])
                plsc.store_scatter(out_v, [rows, cols[j]], v)
            nv = plsc.load_gather(nu_v, [rows])
            plsc.store_scatter(out_v, [rows, cols[_ED]], nv)
            lv = plsc.load_gather(l_v, [rows])
            plsc.store_scatter(out_v, [rows, cols[_ED + 1]], lv)
            for j in range(_LL):
                v = plsc.load_gather(ll_v, [rows, llc[j]])
                plsc.store_scatter(out_v, [rows, cols[_ED + 2 + j]], v)

        out_descs[k] = pltpu.async_copy(out_v, out_hbm.at[pl.ds(base, _CNT)], sem_o)

    for d in out_descs:
        d.wait()


@jax.jit
def _node_feat(an, nu, l, ll, tab):
    mesh = plsc.VectorSubcoreMesh(core_axis_name="c", subcore_axis_name="s")
    run = pl.kernel(
        _body,
        out_type=jax.ShapeDtypeStruct((_N, _OUT_D), jnp.float32),
        mesh=mesh,
        scratch_types=[
            pltpu.VMEM((_CNT,), jnp.int32),
            pltpu.VMEM((_CNT,), jnp.float32),
            pltpu.VMEM((_CNT,), jnp.float32),
            pltpu.VMEM((_CNT, _LL), jnp.float32),
            pltpu.VMEM((10, _ED), jnp.float32),
            [pltpu.VMEM((_CNT, _OUT_D), jnp.float32) for _ in range(_NC)],
            pltpu.SemaphoreType.DMA,
            pltpu.SemaphoreType.DMA,
        ],
        compiler_params=pltpu.CompilerParams(
            use_tc_tiling_on_sc=False, needs_layout_passes=False
        ),
    )
    return run(an, nu, l, ll, tab)


def kernel(atomic_numbers, max_nu, max_l, l_list, emb_table):
    return _node_feat(atomic_numbers, max_nu, max_l, l_list, emb_table)
